# baseline (speedup 1.0000x reference)
import functools

import jax
import jax.numpy as jnp
from jax import lax
from jax.experimental import pallas as pl
from jax.experimental.pallas import tpu as pltpu
from jax.experimental.pallas import tpu_sc as plsc

N_NODES = 10000
N_PAD = 10240
D = 128
DH = 64
NC = 2
NT = 16
CH = 40
SUP = 25
BN = 1000
BE = 2000


def _node_mm_body(h_ref, w_ref, b_ref, srct0_ref, srct1_ref, dstt_ref, ah_ref):
    hw = (jnp.dot(h_ref[...], w_ref[...], preferred_element_type=jnp.float32)
          + b_ref[...])
    srct0_ref[...] = hw[:, :D]
    srct1_ref[...] = hw[:, D:2 * D]
    dstt_ref[...] = hw[:, 2 * D:3 * D]
    ah_ref[...] = hw[:, 3 * D:]


def _node_mm(h, wcat, bcat):
    one = jax.ShapeDtypeStruct((N_NODES, D), jnp.float32)
    return pl.pallas_call(
        _node_mm_body,
        grid=(N_NODES // BN,),
        in_specs=[
            pl.BlockSpec((BN, D), lambda i: (i, 0)),
            pl.BlockSpec((D, 4 * D), lambda i: (0, 0)),
            pl.BlockSpec((4 * D,), lambda i: (0,)),
        ],
        out_specs=[pl.BlockSpec((BN, D), lambda i: (i, 0))] * 4,
        out_shape=[one, one, one, one],
    )(h, wcat, bcat)


def _edge_mm_body(e_ref, cw_ref, cb_ref, ce0_ref, ce1_ref):
    ce = (jnp.dot(e_ref[...], cw_ref[...], preferred_element_type=jnp.float32)
          + cb_ref[...])
    ce0_ref[...] = ce[:, :DH]
    ce1_ref[...] = ce[:, DH:]


def _edge_mm(e, cw, cb, row0, n_rows):
    half = jax.ShapeDtypeStruct((n_rows, DH), jnp.float32)
    blk0 = row0 // BE
    return pl.pallas_call(
        _edge_mm_body,
        grid=(n_rows // BE,),
        in_specs=[
            pl.BlockSpec((BE, D), lambda i: (blk0 + i, 0)),
            pl.BlockSpec((D, D), lambda i: (0, 0)),
            pl.BlockSpec((D,), lambda i: (0,)),
        ],
        out_specs=[pl.BlockSpec((BE, DH), lambda i: (i, 0))] * 2,
        out_shape=[half, half],
    )(e, cw, cb)


def _node_mm_fused(h, ah, wcat, bcat, *nds):
    n_nd = len(nds)

    def body(h_ref, ah_ref, w_ref, b_ref, *refs):
        nd = refs[0][...]
        for r in refs[1:n_nd]:
            nd = nd + r[...]
        srct0_ref, srct1_ref, dstt_ref, ah_ref_out, hout_ref = refs[n_nd:]
        num = jnp.concatenate([nd[0, :, :DH], nd[1, :, :DH]], axis=1)
        den = jnp.concatenate([nd[0, :, DH:], nd[1, :, DH:]], axis=1)
        h1 = h_ref[...] + jnp.maximum(ah_ref[...] + num / (den + 1e-6), 0.0)
        hout_ref[...] = h1
        hw = (jnp.dot(h1, w_ref[...], preferred_element_type=jnp.float32)
              + b_ref[...])
        srct0_ref[...] = hw[:, :D]
        srct1_ref[...] = hw[:, D:2 * D]
        dstt_ref[...] = hw[:, 2 * D:3 * D]
        ah_ref_out[...] = hw[:, 3 * D:]

    one = jax.ShapeDtypeStruct((N_NODES, D), jnp.float32)
    return pl.pallas_call(
        body,
        grid=(N_NODES // BN,),
        in_specs=[
            pl.BlockSpec((BN, D), lambda i: (i, 0)),
            pl.BlockSpec((BN, D), lambda i: (i, 0)),
            pl.BlockSpec((D, 4 * D), lambda i: (0, 0)),
            pl.BlockSpec((4 * D,), lambda i: (0,)),
        ] + [pl.BlockSpec((NC, BN, D), lambda i: (0, i, 0))] * n_nd,
        out_specs=[pl.BlockSpec((BN, D), lambda i: (i, 0))] * 5,
        out_shape=[one, one, one, one, one],
    )(h, ah, wcat, bcat, *nds)


def _edge_mm_fused_body(e_ref, eh_ref, cw_ref, cb_ref,
                        ce0_ref, ce1_ref, eout_ref):
    eh = eh_ref[...]
    ehat = jnp.concatenate([eh[0], eh[1]], axis=1)
    e1 = e_ref[...] + jnp.maximum(ehat, 0.0)
    eout_ref[...] = e1
    ce = (jnp.dot(e1, cw_ref[...], preferred_element_type=jnp.float32)
          + cb_ref[...])
    ce0_ref[...] = ce[:, :DH]
    ce1_ref[...] = ce[:, DH:]


def _edge_mm_fused(e, ehat, cw, cb, n_rows):
    half = jax.ShapeDtypeStruct((n_rows, DH), jnp.float32)
    return pl.pallas_call(
        _edge_mm_fused_body,
        grid=(n_rows // BE,),
        in_specs=[
            pl.BlockSpec((BE, D), lambda i: (i, 0)),
            pl.BlockSpec((NC, BE, DH), lambda i: (0, i, 0)),
            pl.BlockSpec((D, D), lambda i: (0, 0)),
            pl.BlockSpec((D,), lambda i: (0,)),
        ],
        out_specs=[pl.BlockSpec((BE, DH), lambda i: (i, 0))] * 2
        + [pl.BlockSpec((BE, D), lambda i: (i, 0))],
        out_shape=[half, half,
                   jax.ShapeDtypeStruct((n_rows, D), jnp.float32)],
    )(e, ehat, cw, cb)


def _h_update(h, ah, *numdens):
    n_nd = len(numdens)

    def body(h_ref, ah_ref, *refs):
        out_ref = refs[-1]
        nd = refs[0][...]
        for r in refs[1:-1]:
            nd = nd + r[...]
        num = jnp.concatenate([nd[0, :, :DH], nd[1, :, :DH]], axis=1)
        den = jnp.concatenate([nd[0, :, DH:], nd[1, :, DH:]], axis=1)
        h_hat = ah_ref[...] + num / (den + 1e-6)
        out_ref[...] = h_ref[...] + jnp.maximum(h_hat, 0.0)

    return pl.pallas_call(
        body,
        grid=(N_NODES // BN,),
        in_specs=[
            pl.BlockSpec((BN, D), lambda i: (i, 0)),
            pl.BlockSpec((BN, D), lambda i: (i, 0)),
        ] + [pl.BlockSpec((NC, BN, D), lambda i: (0, i, 0))] * n_nd,
        out_specs=pl.BlockSpec((BN, D), lambda i: (i, 0)),
        out_shape=jax.ShapeDtypeStruct((N_NODES, D), jnp.float32),
    )(h, ah, *numdens)


def _e_update_body(e_ref, eh_ref, out_ref):
    eh = eh_ref[...]
    ehat = jnp.concatenate([eh[0], eh[1]], axis=1)
    out_ref[...] = e_ref[...] + jnp.maximum(ehat, 0.0)


def _e_update(e, ehat, n_edges):
    return pl.pallas_call(
        _e_update_body,
        grid=(n_edges // BE,),
        in_specs=[
            pl.BlockSpec((BE, D), lambda i: (i, 0)),
            pl.BlockSpec((NC, BE, DH), lambda i: (0, i, 0)),
        ],
        out_specs=pl.BlockSpec((BE, D), lambda i: (i, 0)),
        out_shape=jax.ShapeDtypeStruct((n_edges, D), jnp.float32),
    )(e, ehat)


def _make_edge_kernel(n_edges, e_keep):
    ept = n_edges // NT
    nch = ept // CH
    nsup = nch // SUP
    rpt = N_PAD // NT
    mesh = plsc.VectorSubcoreMesh(core_axis_name="c", subcore_axis_name="s")

    @functools.partial(
        pl.kernel,
        out_type=[
            jax.ShapeDtypeStruct((NC, max(e_keep, 8), DH), jnp.float32),
            jax.ShapeDtypeStruct((NC, N_PAD, D), jnp.float32),
        ],
        mesh=mesh,
        scratch_types=[
            pltpu.VMEM((CH, D), jnp.float32),
            pltpu.VMEM((CH, D), jnp.float32),
            pltpu.VMEM((CH, D), jnp.float32),
            pltpu.VMEM((CH, D), jnp.float32),
            pltpu.VMEM((CH, DH), jnp.float32),
            pltpu.VMEM((CH, DH), jnp.float32),
            pltpu.VMEM((SUP, CH), jnp.int32),
            pltpu.VMEM((SUP, CH), jnp.int32),
            pltpu.VMEM_SHARED((N_PAD, D), jnp.float32),
            pltpu.SemaphoreType.DMA,
            pltpu.SemaphoreType.DMA,
            pltpu.SemaphoreType.DMA,
            pltpu.SemaphoreType.DMA,
            pltpu.SemaphoreType.DMA,
            pltpu.SemaphoreType.DMA,
        ],
    )
    def edge_kernel(srct0, srct1, dstt, ce0, ce1, srcb, dstr, ehat, numden,
                    sbuf0, sbuf1, dbuf0, dbuf1, cbuf0, cbuf1,
                    sidx_sup, didx_sup, acc,
                    ss0, ss1, sd0, sd1, sc0, sc1):
        c = lax.axis_index("c")
        s = lax.axis_index("s")
        sbuf = (sbuf0, sbuf1)
        dbuf = (dbuf0, dbuf1)
        cbuf = (cbuf0, cbuf1)
        ssem = (ss0, ss1)
        dsem = (sd0, sd1)
        csem = (sc0, sc1)
        srct = (srct0, srct1)
        cet = (ce0, ce1)

        def zrow(i, carry):
            for q in range(D // 16):
                sbuf0[i, pl.ds(q * 16, 16)] = jnp.zeros((16,), jnp.float32)
            return carry
        lax.fori_loop(0, CH, zrow, 0)
        r0 = s * rpt
        for b in range(rpt // CH):
            pltpu.sync_copy(sbuf0, acc.at[pl.ds(r0 + b * CH, CH)])
        plsc.subcore_barrier()

        base0 = s * ept
        write_ehat = base0 < e_keep

        def issue(g, kk, p):
            base = base0 + (g * SUP + kk) * CH
            for cc in range(NC):
                @pl.when(c == cc)
                def _():
                    pltpu.async_copy(srct[cc].at[sidx_sup.at[kk]],
                                     sbuf[p], ssem[p])
                    pltpu.async_copy(cet[cc].at[pl.ds(base, CH)],
                                     cbuf[p], csem[p])
            pltpu.async_copy(dstt.at[didx_sup.at[kk]], dbuf[p], dsem[p])

        def drain(g, kk, p):
            base = base0 + (g * SUP + kk) * CH
            for cc in range(NC):
                @pl.when(c == cc)
                def _():
                    pltpu.make_async_copy(srct[cc].at[sidx_sup.at[kk]],
                                          sbuf[p], ssem[p]).wait()
                    pltpu.make_async_copy(cet[cc].at[pl.ds(base, CH)],
                                          cbuf[p], csem[p]).wait()
            pltpu.make_async_copy(dstt.at[didx_sup.at[kk]], dbuf[p], dsem[p]).wait()

            def rows(col0, store_eh):
                def row(j):
                    for q in range(DH // 16):
                        sl = pl.ds(q * 16, 16)
                        sl2 = pl.ds(DH + q * 16, 16)
                        bv = sbuf[p][j, sl]
                        dv = sbuf[p][j, sl2]
                        eh = (cbuf[p][j, sl] + dv
                              + dbuf[p][j, pl.ds(col0 + q * 16, 16)])
                        if store_eh:
                            cbuf[p][j, sl] = eh
                        sg = 1.0 / (1.0 + jnp.exp(-eh))
                        sbuf[p][j, sl] = sg * bv
                        sbuf[p][j, sl2] = sg

                def row8(j8, rcarry):
                    for u in range(8):
                        row(8 * j8 + u)
                    return rcarry
                lax.fori_loop(0, CH // 8, row8, 0)

            for cc, col0 in ((0, 0), (1, DH)):
                @pl.when((c == cc) & write_ehat)
                def _(col0=col0):
                    rows(col0, True)

                @pl.when((c == cc) & jnp.logical_not(write_ehat))
                def _(col0=col0):
                    rows(col0, False)

            @pl.when(write_ehat)
            def _():
                pltpu.sync_copy(cbuf[p], ehat.at[c, pl.ds(base, CH)])

            pltpu.sync_copy(sbuf[p], acc.at[didx_sup.at[kk]], add=True)

        def superstep(g, carry):
            pltpu.sync_copy(srcb.at[s, g], sidx_sup)
            pltpu.sync_copy(dstr.at[s, g], didx_sup)
            issue(g, 0, 0)

            def pair(kkp, pcarry):
                kk0 = 2 * kkp
                kk1 = kk0 + 1
                issue(g, kk1, 1)
                drain(g, kk0, 0)

                @pl.when(kk0 + 2 < SUP)
                def _():
                    issue(g, kk0 + 2, 0)
                drain(g, kk1, 1)
                return pcarry
            lax.fori_loop(0, SUP // 2, pair, 0)
            if SUP % 2:
                drain(g, SUP - 1, 0)
            return carry
        lax.fori_loop(0, nsup, superstep, 0)

        plsc.subcore_barrier()
        pltpu.sync_copy(acc.at[pl.ds(r0, rpt)], numden.at[c, pl.ds(r0, rpt)])

    return edge_kernel


_EDGE_KERNELS = {}


def _edge_kernel_for(n_edges, e_keep):
    key = (n_edges, e_keep)
    if key not in _EDGE_KERNELS:
        _EDGE_KERNELS[key] = _make_edge_kernel(n_edges, e_keep)
    return _EDGE_KERNELS[key]


def _packed_weights(A_w, A_b, B_w, B_b, D_w, D_b, E_w, E_b, i):
    wcat = jnp.concatenate([
        B_w[i][:, :DH], D_w[i][:, :DH],
        B_w[i][:, DH:], D_w[i][:, DH:],
        E_w[i], A_w[i],
    ], axis=1)
    bcat = jnp.concatenate([
        B_b[i][:DH], D_b[i][:DH],
        B_b[i][DH:], D_b[i][DH:],
        E_b[i], A_b[i],
    ])
    return wcat, bcat


def _sc_edge_pass(ei, edge0, n_edges, e_keep, srct0, srct1, dstt, ce0, ce1):
    nsup = n_edges // NT // CH // SUP
    srcb = lax.slice(ei[0], (edge0,), (edge0 + n_edges,)).reshape(
        NT, nsup, SUP, CH)
    dstr = lax.slice(ei[1], (edge0,), (edge0 + n_edges,)).reshape(
        NT, nsup, SUP, CH)
    return _edge_kernel_for(n_edges, e_keep)(
        srct0, srct1, dstt, ce0, ce1, srcb, dstr)


def kernel(h, e, edge_index0, edge_index1, A_w, A_b, B_w, B_b, C_w, C_b,
           D_w, D_b, E_w, E_b):
    n0 = edge_index0.shape[1]
    n1 = edge_index1.shape[1]
    e_keep0 = min(n1, n0)
    h0a = n0 // 2
    h1a = n1 // 2

    wcat0, bcat0 = _packed_weights(A_w, A_b, B_w, B_b, D_w, D_b, E_w, E_b, 0)
    srct0, srct1, dstt, ah0 = _node_mm(h, wcat0, bcat0)
    ce0a, ce1a = _edge_mm(e, C_w[0], C_b[0], 0, h0a)
    ce0b, ce1b = _edge_mm(e, C_w[0], C_b[0], h0a, n0 - h0a)
    ehat0, nd0a = _sc_edge_pass(edge_index0, 0, h0a, min(e_keep0, h0a),
                                srct0, srct1, dstt, ce0a, ce1a)
    _, nd0b = _sc_edge_pass(edge_index0, h0a, n0 - h0a,
                            max(e_keep0 - h0a, 0),
                            srct0, srct1, dstt, ce0b, ce1b)

    wcat1, bcat1 = _packed_weights(A_w, A_b, B_w, B_b, D_w, D_b, E_w, E_b, 1)
    ce0, ce1, e1 = _edge_mm_fused(e[:e_keep0], ehat0, C_w[1], C_b[1], n1)
    srct0, srct1, dstt, ah1, h1 = _node_mm_fused(h, ah0, wcat1, bcat1,
                                                 nd0a, nd0b)
    ehat1, nd1 = _sc_edge_pass(edge_index1, 0, n1, n1,
                               srct0, srct1, dstt, ce0, ce1)

    h2 = _h_update(h1, ah1, nd1)
    e2 = _e_update(e1, ehat1, n1)
    return (h2, e2)


# speedup vs baseline: 1.0041x; 1.0041x over previous
import functools

import jax
import jax.numpy as jnp
from jax import lax
from jax.experimental import pallas as pl
from jax.experimental.pallas import tpu as pltpu
from jax.experimental.pallas import tpu_sc as plsc

N_NODES = 10000
N_PAD = 10240
D = 128
DH = 64
NC = 2
NT = 16
CH = 40
SUP = 25
BN = 1000
BE = 2000


def _node_mm_body(h_ref, w_ref, b_ref, srct0_ref, srct1_ref, dstt_ref, ah_ref):
    hw = (jnp.dot(h_ref[...], w_ref[...], preferred_element_type=jnp.float32)
          + b_ref[...])
    srct0_ref[...] = hw[:, :D]
    srct1_ref[...] = hw[:, D:2 * D]
    dstt_ref[...] = hw[:, 2 * D:3 * D]
    ah_ref[...] = hw[:, 3 * D:]


def _node_mm(h, wcat, bcat):
    one = jax.ShapeDtypeStruct((N_NODES, D), jnp.float32)
    return pl.pallas_call(
        _node_mm_body,
        grid=(N_NODES // BN,),
        in_specs=[
            pl.BlockSpec((BN, D), lambda i: (i, 0)),
            pl.BlockSpec((D, 4 * D), lambda i: (0, 0)),
            pl.BlockSpec((4 * D,), lambda i: (0,)),
        ],
        out_specs=[pl.BlockSpec((BN, D), lambda i: (i, 0))] * 4,
        out_shape=[one, one, one, one],
    )(h, wcat, bcat)


def _edge_mm_body(e_ref, cw_ref, cb_ref, ce0_ref, ce1_ref):
    ce = (jnp.dot(e_ref[...], cw_ref[...], preferred_element_type=jnp.float32)
          + cb_ref[...])
    ce0_ref[...] = ce[:, :DH]
    ce1_ref[...] = ce[:, DH:]


def _edge_mm(e, cw, cb, row0, n_rows):
    half = jax.ShapeDtypeStruct((n_rows, DH), jnp.float32)
    blk0 = row0 // BE
    return pl.pallas_call(
        _edge_mm_body,
        grid=(n_rows // BE,),
        in_specs=[
            pl.BlockSpec((BE, D), lambda i: (blk0 + i, 0)),
            pl.BlockSpec((D, D), lambda i: (0, 0)),
            pl.BlockSpec((D,), lambda i: (0,)),
        ],
        out_specs=[pl.BlockSpec((BE, DH), lambda i: (i, 0))] * 2,
        out_shape=[half, half],
    )(e, cw, cb)


def _node_mm_fused(h, ah, wcat, bcat, *nds):
    n_nd = len(nds)

    def body(h_ref, ah_ref, w_ref, b_ref, *refs):
        nd = refs[0][...]
        for r in refs[1:n_nd]:
            nd = nd + r[...]
        srct0_ref, srct1_ref, dstt_ref, ah_ref_out, hout_ref = refs[n_nd:]
        num = jnp.concatenate([nd[0, :, :DH], nd[1, :, :DH]], axis=1)
        den = jnp.concatenate([nd[0, :, DH:], nd[1, :, DH:]], axis=1)
        h1 = h_ref[...] + jnp.maximum(ah_ref[...] + num / (den + 1e-6), 0.0)
        hout_ref[...] = h1
        hw = (jnp.dot(h1, w_ref[...], preferred_element_type=jnp.float32)
              + b_ref[...])
        srct0_ref[...] = hw[:, :D]
        srct1_ref[...] = hw[:, D:2 * D]
        dstt_ref[...] = hw[:, 2 * D:3 * D]
        ah_ref_out[...] = hw[:, 3 * D:]

    one = jax.ShapeDtypeStruct((N_NODES, D), jnp.float32)
    return pl.pallas_call(
        body,
        grid=(N_NODES // BN,),
        in_specs=[
            pl.BlockSpec((BN, D), lambda i: (i, 0)),
            pl.BlockSpec((BN, D), lambda i: (i, 0)),
            pl.BlockSpec((D, 4 * D), lambda i: (0, 0)),
            pl.BlockSpec((4 * D,), lambda i: (0,)),
        ] + [pl.BlockSpec((NC, BN, D), lambda i: (0, i, 0))] * n_nd,
        out_specs=[pl.BlockSpec((BN, D), lambda i: (i, 0))] * 5,
        out_shape=[one, one, one, one, one],
    )(h, ah, wcat, bcat, *nds)


def _edge_mm_fused_body(e_ref, eh_ref, cw_ref, cb_ref,
                        ce0_ref, ce1_ref, eout_ref):
    eh = eh_ref[...]
    ehat = jnp.concatenate([eh[0], eh[1]], axis=1)
    e1 = e_ref[...] + jnp.maximum(ehat, 0.0)
    eout_ref[...] = e1
    ce = (jnp.dot(e1, cw_ref[...], preferred_element_type=jnp.float32)
          + cb_ref[...])
    ce0_ref[...] = ce[:, :DH]
    ce1_ref[...] = ce[:, DH:]


def _edge_mm_fused(e, ehat, cw, cb, n_rows):
    half = jax.ShapeDtypeStruct((n_rows, DH), jnp.float32)
    return pl.pallas_call(
        _edge_mm_fused_body,
        grid=(n_rows // BE,),
        in_specs=[
            pl.BlockSpec((BE, D), lambda i: (i, 0)),
            pl.BlockSpec((NC, BE, DH), lambda i: (0, i, 0)),
            pl.BlockSpec((D, D), lambda i: (0, 0)),
            pl.BlockSpec((D,), lambda i: (0,)),
        ],
        out_specs=[pl.BlockSpec((BE, DH), lambda i: (i, 0))] * 2
        + [pl.BlockSpec((BE, D), lambda i: (i, 0))],
        out_shape=[half, half,
                   jax.ShapeDtypeStruct((n_rows, D), jnp.float32)],
    )(e, ehat, cw, cb)


def _h_update(h, ah, *numdens):
    n_nd = len(numdens)

    def body(h_ref, ah_ref, *refs):
        out_ref = refs[-1]
        nd = refs[0][...]
        for r in refs[1:-1]:
            nd = nd + r[...]
        num = jnp.concatenate([nd[0, :, :DH], nd[1, :, :DH]], axis=1)
        den = jnp.concatenate([nd[0, :, DH:], nd[1, :, DH:]], axis=1)
        h_hat = ah_ref[...] + num / (den + 1e-6)
        out_ref[...] = h_ref[...] + jnp.maximum(h_hat, 0.0)

    return pl.pallas_call(
        body,
        grid=(N_NODES // BN,),
        in_specs=[
            pl.BlockSpec((BN, D), lambda i: (i, 0)),
            pl.BlockSpec((BN, D), lambda i: (i, 0)),
        ] + [pl.BlockSpec((NC, BN, D), lambda i: (0, i, 0))] * n_nd,
        out_specs=pl.BlockSpec((BN, D), lambda i: (i, 0)),
        out_shape=jax.ShapeDtypeStruct((N_NODES, D), jnp.float32),
    )(h, ah, *numdens)


def _e_update_body(e_ref, eh_ref, out_ref):
    eh = eh_ref[...]
    ehat = jnp.concatenate([eh[0], eh[1]], axis=1)
    out_ref[...] = e_ref[...] + jnp.maximum(ehat, 0.0)


def _e_update(e, ehat, n_edges):
    return pl.pallas_call(
        _e_update_body,
        grid=(n_edges // BE,),
        in_specs=[
            pl.BlockSpec((BE, D), lambda i: (i, 0)),
            pl.BlockSpec((NC, BE, DH), lambda i: (0, i, 0)),
        ],
        out_specs=pl.BlockSpec((BE, D), lambda i: (i, 0)),
        out_shape=jax.ShapeDtypeStruct((n_edges, D), jnp.float32),
    )(e, ehat)


def _make_edge_kernel(n_edges, e_keep):
    ept = n_edges // NT
    nch = ept // CH
    nsup = nch // SUP
    rpt = N_PAD // NT
    mesh = plsc.VectorSubcoreMesh(core_axis_name="c", subcore_axis_name="s")

    @functools.partial(
        pl.kernel,
        out_type=[
            jax.ShapeDtypeStruct((NC, max(e_keep, 8), DH), jnp.float32),
            jax.ShapeDtypeStruct((NC, N_PAD, D), jnp.float32),
        ],
        mesh=mesh,
        scratch_types=[
            pltpu.VMEM((CH, D), jnp.float32),
            pltpu.VMEM((CH, D), jnp.float32),
            pltpu.VMEM((CH, D), jnp.float32),
            pltpu.VMEM((CH, D), jnp.float32),
            pltpu.VMEM((CH, DH), jnp.float32),
            pltpu.VMEM((CH, DH), jnp.float32),
            pltpu.VMEM((SUP, CH), jnp.int32),
            pltpu.VMEM((SUP, CH), jnp.int32),
            pltpu.VMEM_SHARED((N_PAD, D), jnp.float32),
            pltpu.SemaphoreType.DMA,
            pltpu.SemaphoreType.DMA,
            pltpu.SemaphoreType.DMA,
            pltpu.SemaphoreType.DMA,
            pltpu.SemaphoreType.DMA,
            pltpu.SemaphoreType.DMA,
        ],
    )
    def edge_kernel(srct0, srct1, dstt, ce0, ce1, srcb, dstr, ehat, numden,
                    sbuf0, sbuf1, dbuf0, dbuf1, cbuf0, cbuf1,
                    sidx_sup, didx_sup, acc,
                    ss0, ss1, sd0, sd1, sc0, sc1):
        c = lax.axis_index("c")
        s = lax.axis_index("s")
        sbuf = (sbuf0, sbuf1)
        dbuf = (dbuf0, dbuf1)
        cbuf = (cbuf0, cbuf1)
        ssem = (ss0, ss1)
        dsem = (sd0, sd1)
        csem = (sc0, sc1)
        srct = (srct0, srct1)
        cet = (ce0, ce1)

        def zrow(i, carry):
            for q in range(D // 16):
                sbuf0[i, pl.ds(q * 16, 16)] = jnp.zeros((16,), jnp.float32)
            return carry
        lax.fori_loop(0, CH, zrow, 0)
        r0 = s * rpt
        for b in range(rpt // CH):
            pltpu.sync_copy(sbuf0, acc.at[pl.ds(r0 + b * CH, CH)])
        plsc.subcore_barrier()

        base0 = s * ept
        write_ehat = base0 < e_keep

        def issue(g, kk, p):
            base = base0 + (g * SUP + kk) * CH
            for cc in range(NC):
                @pl.when(c == cc)
                def _():
                    pltpu.async_copy(srct[cc].at[sidx_sup.at[kk]],
                                     sbuf[p], ssem[p])
                    pltpu.async_copy(cet[cc].at[pl.ds(base, CH)],
                                     cbuf[p], csem[p])
            pltpu.async_copy(dstt.at[didx_sup.at[kk]], dbuf[p], dsem[p])

        def drain(g, kk, p):
            base = base0 + (g * SUP + kk) * CH
            for cc in range(NC):
                @pl.when(c == cc)
                def _():
                    pltpu.make_async_copy(srct[cc].at[sidx_sup.at[kk]],
                                          sbuf[p], ssem[p]).wait()
                    pltpu.make_async_copy(cet[cc].at[pl.ds(base, CH)],
                                          cbuf[p], csem[p]).wait()
            pltpu.make_async_copy(dstt.at[didx_sup.at[kk]], dbuf[p], dsem[p]).wait()

            def rows(col0, store_eh):
                def row(j):
                    for q in range(DH // 16):
                        sl = pl.ds(q * 16, 16)
                        sl2 = pl.ds(DH + q * 16, 16)
                        bv = sbuf[p][j, sl]
                        dv = sbuf[p][j, sl2]
                        eh = (cbuf[p][j, sl] + dv
                              + dbuf[p][j, pl.ds(col0 + q * 16, 16)])
                        if store_eh:
                            cbuf[p][j, sl] = eh
                        sg = 1.0 / (1.0 + jnp.exp(-eh))
                        sbuf[p][j, sl] = sg * bv
                        sbuf[p][j, sl2] = sg

                def row4(j4, rcarry):
                    for u in range(4):
                        row(4 * j4 + u)
                    return rcarry
                lax.fori_loop(0, CH // 4, row4, 0)

            for cc, col0 in ((0, 0), (1, DH)):
                @pl.when((c == cc) & write_ehat)
                def _(col0=col0):
                    rows(col0, True)

                @pl.when((c == cc) & jnp.logical_not(write_ehat))
                def _(col0=col0):
                    rows(col0, False)

            @pl.when(write_ehat)
            def _():
                pltpu.sync_copy(cbuf[p], ehat.at[c, pl.ds(base, CH)])

            pltpu.sync_copy(sbuf[p], acc.at[didx_sup.at[kk]], add=True)

        def superstep(g, carry):
            pltpu.sync_copy(srcb.at[s, g], sidx_sup)
            pltpu.sync_copy(dstr.at[s, g], didx_sup)
            issue(g, 0, 0)

            def pair(kkp, pcarry):
                kk0 = 2 * kkp
                kk1 = kk0 + 1
                issue(g, kk1, 1)
                drain(g, kk0, 0)

                @pl.when(kk0 + 2 < SUP)
                def _():
                    issue(g, kk0 + 2, 0)
                drain(g, kk1, 1)
                return pcarry
            lax.fori_loop(0, SUP // 2, pair, 0)
            if SUP % 2:
                drain(g, SUP - 1, 0)
            return carry
        lax.fori_loop(0, nsup, superstep, 0)

        plsc.subcore_barrier()
        pltpu.sync_copy(acc.at[pl.ds(r0, rpt)], numden.at[c, pl.ds(r0, rpt)])

    return edge_kernel


_EDGE_KERNELS = {}


def _edge_kernel_for(n_edges, e_keep):
    key = (n_edges, e_keep)
    if key not in _EDGE_KERNELS:
        _EDGE_KERNELS[key] = _make_edge_kernel(n_edges, e_keep)
    return _EDGE_KERNELS[key]


def _packed_weights(A_w, A_b, B_w, B_b, D_w, D_b, E_w, E_b, i):
    wcat = jnp.concatenate([
        B_w[i][:, :DH], D_w[i][:, :DH],
        B_w[i][:, DH:], D_w[i][:, DH:],
        E_w[i], A_w[i],
    ], axis=1)
    bcat = jnp.concatenate([
        B_b[i][:DH], D_b[i][:DH],
        B_b[i][DH:], D_b[i][DH:],
        E_b[i], A_b[i],
    ])
    return wcat, bcat


def _sc_edge_pass(ei, edge0, n_edges, e_keep, srct0, srct1, dstt, ce0, ce1):
    nsup = n_edges // NT // CH // SUP
    srcb = lax.slice(ei[0], (edge0,), (edge0 + n_edges,)).reshape(
        NT, nsup, SUP, CH)
    dstr = lax.slice(ei[1], (edge0,), (edge0 + n_edges,)).reshape(
        NT, nsup, SUP, CH)
    return _edge_kernel_for(n_edges, e_keep)(
        srct0, srct1, dstt, ce0, ce1, srcb, dstr)


def kernel(h, e, edge_index0, edge_index1, A_w, A_b, B_w, B_b, C_w, C_b,
           D_w, D_b, E_w, E_b):
    n0 = edge_index0.shape[1]
    n1 = edge_index1.shape[1]
    e_keep0 = min(n1, n0)
    h0a = n0 // 2
    h1a = n1 // 2

    wcat0, bcat0 = _packed_weights(A_w, A_b, B_w, B_b, D_w, D_b, E_w, E_b, 0)
    srct0, srct1, dstt, ah0 = _node_mm(h, wcat0, bcat0)
    ce0a, ce1a = _edge_mm(e, C_w[0], C_b[0], 0, h0a)
    ce0b, ce1b = _edge_mm(e, C_w[0], C_b[0], h0a, n0 - h0a)
    ehat0, nd0a = _sc_edge_pass(edge_index0, 0, h0a, min(e_keep0, h0a),
                                srct0, srct1, dstt, ce0a, ce1a)
    _, nd0b = _sc_edge_pass(edge_index0, h0a, n0 - h0a,
                            max(e_keep0 - h0a, 0),
                            srct0, srct1, dstt, ce0b, ce1b)

    wcat1, bcat1 = _packed_weights(A_w, A_b, B_w, B_b, D_w, D_b, E_w, E_b, 1)
    ce0, ce1, e1 = _edge_mm_fused(e[:e_keep0], ehat0, C_w[1], C_b[1], n1)
    srct0, srct1, dstt, ah1, h1 = _node_mm_fused(h, ah0, wcat1, bcat1,
                                                 nd0a, nd0b)
    ehat1, nd1 = _sc_edge_pass(edge_index1, 0, n1, n1,
                               srct0, srct1, dstt, ce0, ce1)

    h2 = _h_update(h1, ah1, nd1)
    e2 = _e_update(e1, ehat1, n1)
    return (h2, e2)
